# baseline (device time: 42784 ns/iter reference)
import jax
import jax.numpy as jnp
from jax import lax
from jax.experimental import pallas as pl
from jax.experimental.pallas import tpu as pltpu

N_DEV = 4
_BF = jnp.bfloat16


def kernel(x, Win0, Wout0, Win1, Wout1, Win2, Wout2):
    B, d_sh = x.shape
    H = Win0.shape[1]
    Bq = B // N_DEV

    def body(x_ref, win0_ref, wout0_ref, win1_ref, wout1_ref, win2_ref,
             wout2_ref, out_ref, p_send, rs_buf, hown_buf, hg_buf,
             send_sems, recv_sems):
        me = lax.axis_index("i")

        bsem = pltpu.get_barrier_semaphore()
        for d in range(1, N_DEV):
            pl.semaphore_signal(
                bsem, inc=1,
                device_id=((me + d) % N_DEV,),
                device_id_type=pl.DeviceIdType.MESH,
            )
        pl.semaphore_wait(bsem, N_DEV - 1)

        win_refs = [win0_ref, win1_ref, win2_ref]
        wout_refs = [wout0_ref, wout1_ref, wout2_ref]
        sends = []

        def rs_send(L, dd, blk_bf):
            tgt = (me + dd) % N_DEV
            p_send[L, dd] = blk_bf
            rdma = pltpu.make_async_remote_copy(
                src_ref=p_send.at[L, dd],
                dst_ref=rs_buf.at[L, N_DEV - dd],
                send_sem=send_sems.at[2 * L, dd],
                recv_sem=recv_sems.at[2 * L, N_DEV - dd],
                device_id=(tgt,),
                device_id_type=pl.DeviceIdType.MESH,
            )
            rdma.start()
            sends.append(rdma)

        def wait_recv(phase, dd, buf_ref):
            recv = pltpu.make_async_remote_copy(
                src_ref=buf_ref, dst_ref=buf_ref,
                send_sem=send_sems.at[phase, dd],
                recv_sem=recv_sems.at[phase, dd],
                device_id=(me,),
                device_id_type=pl.DeviceIdType.MESH,
            )
            recv.wait_recv()

        w_in0 = win_refs[0][...].astype(_BF)
        for dd in (2, 1, 3):
            s = (me + dd) % N_DEV
            blk = jnp.dot(
                x_ref[pl.ds(s * Bq, Bq), :].astype(_BF), w_in0,
                preferred_element_type=jnp.float32,
            ).astype(_BF)
            rs_send(0, dd, blk)
        acc = jnp.dot(
            x_ref[pl.ds(me * Bq, Bq), :].astype(_BF), w_in0,
            preferred_element_type=jnp.float32,
        )

        for L in range(3):
            p_ag = 2 * L + 1
            w_out = wout_refs[L][...].astype(_BF)
            w_in_next = win_refs[L + 1][...].astype(_BF) if L < 2 else None

            for dd in range(1, N_DEV):
                wait_recv(2 * L, dd, rs_buf.at[L, dd])
                acc = acc + rs_buf[L, dd].astype(jnp.float32)
            hown_buf[L] = jnp.maximum(acc, 0.0).astype(_BF)

            for dd in (2, 1, 3):
                tgt = (me + dd) % N_DEV
                rdma = pltpu.make_async_remote_copy(
                    src_ref=hown_buf.at[L],
                    dst_ref=hg_buf.at[L, N_DEV - dd],
                    send_sem=send_sems.at[p_ag, dd],
                    recv_sem=recv_sems.at[p_ag, N_DEV - dd],
                    device_id=(tgt,),
                    device_id_type=pl.DeviceIdType.MESH,
                )
                rdma.start()
                sends.append(rdma)

            xnext_own = jnp.dot(
                hown_buf[L], w_out, preferred_element_type=jnp.float32
            )
            if L < 2:
                acc_next = jnp.dot(
                    xnext_own.astype(_BF), w_in_next,
                    preferred_element_type=jnp.float32,
                )
            else:
                out_ref[pl.ds(me * Bq, Bq), :] = xnext_own

            for dd in range(1, N_DEV):
                wait_recv(p_ag, dd, hg_buf.at[L, dd])
                s = (me + dd) % N_DEV
                xnext_s = jnp.dot(
                    hg_buf[L, dd], w_out, preferred_element_type=jnp.float32
                )
                if L < 2:
                    blk = jnp.dot(
                        xnext_s.astype(_BF), w_in_next,
                        preferred_element_type=jnp.float32,
                    ).astype(_BF)
                    rs_send(L + 1, dd, blk)
                else:
                    out_ref[pl.ds(s * Bq, Bq), :] = xnext_s

            if L < 2:
                acc = acc_next

        for rdma in sends:
            rdma.wait_send()

    return pl.pallas_call(
        body,
        out_shape=jax.ShapeDtypeStruct((B, d_sh), jnp.float32),
        in_specs=[pl.BlockSpec(memory_space=pltpu.VMEM)] * 7,
        out_specs=pl.BlockSpec(memory_space=pltpu.VMEM),
        scratch_shapes=[
            pltpu.VMEM((3, N_DEV, Bq, H), _BF),
            pltpu.VMEM((3, N_DEV, Bq, H), _BF),
            pltpu.VMEM((3, Bq, H), _BF),
            pltpu.VMEM((3, N_DEV, Bq, H), _BF),
            pltpu.SemaphoreType.DMA((6, N_DEV)),
            pltpu.SemaphoreType.DMA((6, N_DEV)),
        ],
        compiler_params=pltpu.CompilerParams(collective_id=0),
    )(x, Win0, Wout0, Win1, Wout1, Win2, Wout2)
